# baseline (speedup 1.0000x reference)
import functools

import jax
import jax.numpy as jnp
from jax import lax
from jax.experimental import pallas as pl
from jax.experimental.pallas import tpu as pltpu
from jax.experimental.pallas import tpu_sc as plsc

_B = 16
_S = 512
_C = 2
_T = 4096
_L = 16
_NB = _T // _L
_BIAS = 64

_mesh = plsc.VectorSubcoreMesh(core_axis_name="c", subcore_axis_name="s")


@functools.partial(
    pl.kernel,
    out_type=jax.ShapeDtypeStruct((_C * _B, _T), jnp.float32),
    mesh=_mesh,
    compiler_params=pltpu.CompilerParams(needs_layout_passes=False),
    scratch_types=[
        pltpu.VMEM((_S + 128,), jnp.int32),
        pltpu.VMEM((_BIAS + _S, _C), jnp.float32),
        pltpu.VMEM((_T,), jnp.int32),
        pltpu.VMEM((_T,), jnp.float32),
        pltpu.SemaphoreType.DMA,
        pltpu.SemaphoreType.DMA,
    ],
)
def _decode(vals_hbm, starts_hbm, out_hbm, starts_v, vals_v, m_v, out_v,
            sem_s, sem_v):
    comp = lax.axis_index("c")
    b = lax.axis_index("s")

    iota = lax.iota(jnp.int32, _L)
    plsc.store_scatter(vals_v, [iota >> 1, iota & 1],
                       jnp.zeros((_L,), jnp.float32))
    cp_starts = pltpu.async_copy(starts_hbm.at[b], starts_v.at[pl.ds(0, _S)],
                                 sem_s)
    cp_vals = pltpu.async_copy(vals_hbm.at[b],
                               vals_v.at[pl.ds(_BIAS, _S), :], sem_v)
    zero = jnp.zeros((_L,), jnp.int32)

    @plsc.parallel_loop(0, _NB, unroll=4)
    def init_body(i):
        m_v[pl.ds(i * _L, _L)] = zero

    cp_starts.wait()
    for p in range(128 // _L):
        starts_v[pl.ds(_S + p * _L, _L)] = jnp.full((_L,), _T, jnp.int32)

    @plsc.parallel_loop(0, _S // _L, unroll=4)
    def scat_body(g):
        base = g * _L
        st = starts_v[pl.ds(base, _L)]
        nxt = plsc.load_gather(starts_v, [iota + (base + 1)])
        vis = nxt > st
        plsc.store_scatter(m_v, [st], iota + (base + _BIAS), mask=vis)

    cp_vals.wait()
    comp_vec = zero + comp

    @plsc.parallel_loop(0, _NB, unroll=4, carry=jnp.int32(0))
    def scan_body(i, carry):
        mv = m_v[pl.ds(i * _L, _L)]
        sc = jnp.maximum(plsc.cummax(mv), carry)
        out_v[pl.ds(i * _L, _L)] = plsc.load_gather(vals_v, [sc, comp_vec])
        return jnp.maximum(carry, jnp.max(mv))

    pltpu.sync_copy(out_v, out_hbm.at[comp * _B + b])


def kernel(segment_values, segment_starts):
    out = _decode(segment_values, segment_starts)
    return out.reshape(_C, _B, _T)


# speedup vs baseline: 1.2740x; 1.2740x over previous
import functools

import jax
import jax.numpy as jnp
from jax import lax
from jax.experimental import pallas as pl
from jax.experimental.pallas import tpu as pltpu
from jax.experimental.pallas import tpu_sc as plsc

_B = 16
_S = 512
_C = 2
_T = 4096
_L = 16
_NB = _T // _L
_BIAS = 64

_mesh = plsc.VectorSubcoreMesh(core_axis_name="c", subcore_axis_name="s")


@functools.partial(
    pl.kernel,
    out_type=jax.ShapeDtypeStruct((_C * _B, _T), jnp.float32),
    mesh=_mesh,
    compiler_params=pltpu.CompilerParams(needs_layout_passes=False),
    scratch_types=[
        pltpu.VMEM((_S + 128,), jnp.int32),
        pltpu.VMEM((_S * _C + 128,), jnp.float32),
        pltpu.VMEM((_T,), jnp.int32),
        pltpu.VMEM((_T,), jnp.float32),
        pltpu.SemaphoreType.DMA,
        pltpu.SemaphoreType.DMA,
    ],
)
def _decode(vals_hbm, starts_hbm, out_hbm, starts_v, vals_v, m_v, out_v,
            sem_s, sem_v):
    comp = lax.axis_index("c")
    b = lax.axis_index("s")

    vals_v[pl.ds(0, _L)] = jnp.zeros((_L,), jnp.float32)
    cp_starts = pltpu.async_copy(starts_hbm.at[b], starts_v.at[pl.ds(0, _S)],
                                 sem_s)
    cp_vals = pltpu.async_copy(vals_hbm.at[b],
                               vals_v.at[pl.ds(_BIAS * _C, _S * _C)], sem_v)

    iota = lax.iota(jnp.int32, _L)
    zero = jnp.zeros((_L,), jnp.int32)

    @plsc.parallel_loop(0, _NB, unroll=4)
    def init_body(i):
        m_v[pl.ds(i * _L, _L)] = zero

    cp_starts.wait()
    for p in range(128 // _L):
        starts_v[pl.ds(_S + p * _L, _L)] = jnp.full((_L,), _T, jnp.int32)

    @plsc.parallel_loop(0, _S // _L, unroll=4)
    def scat_body(g):
        base = g * _L
        st = starts_v[pl.ds(base, _L)]
        nxt = plsc.load_gather(starts_v, [iota + (base + 1)])
        vis = nxt > st
        plsc.store_scatter(m_v, [st], iota + (base + _BIAS), mask=vis)

    cp_vals.wait()

    @plsc.parallel_loop(0, _NB, unroll=4, carry=jnp.int32(0))
    def scan_body(i, carry):
        mv = m_v[pl.ds(i * _L, _L)]
        sc = jnp.maximum(plsc.cummax(mv), carry)
        idx = sc * _C + comp
        out_v[pl.ds(i * _L, _L)] = plsc.load_gather(vals_v, [idx])
        return jnp.maximum(carry, jnp.max(mv))

    pltpu.sync_copy(out_v, out_hbm.at[comp * _B + b])


def kernel(segment_values, segment_starts):
    vals_flat = segment_values.reshape(_B, _S * _C)
    out = _decode(vals_flat, segment_starts)
    return out.reshape(_C, _B, _T)
